# initial kernel scaffold (unmeasured)
import jax
import jax.numpy as jnp
from jax import lax
from jax.experimental import pallas as pl
from jax.experimental.pallas import tpu as pltpu

T = 2048
D = 1024


def kernel(ids, E):
    V = E.shape[0]

    def body(ids_smem, ids_vmem, e_hbm, out_ref, send_ref, recv_ref,
             gather_sem, send_sem, recv_sem):
        my_x = lax.axis_index("x")
        my_y = lax.axis_index("y")
        my_z = lax.axis_index("z")
        partner = (1 - my_x, my_y, my_z)

        barrier = pltpu.get_barrier_semaphore()
        pl.semaphore_signal(barrier, inc=1, device_id=partner,
                            device_id_type=pl.DeviceIdType.MESH)

        offset = my_x * V

        def issue(t, carry):
            local = ids_smem[t, 0] - offset
            c = jnp.clip(local, 0, V - 1)
            pltpu.make_async_copy(
                e_hbm.at[pl.ds(c, 1), :],
                out_ref.at[pl.ds(t, 1), :],
                gather_sem,
            ).start()
            return carry

        lax.fori_loop(0, T, issue, 0)

        def drain(t, carry):
            pltpu.make_async_copy(
                e_hbm.at[pl.ds(0, 1), :],
                out_ref.at[pl.ds(t, 1), :],
                gather_sem,
            ).wait()
            return carry

        lax.fori_loop(0, T, drain, 0)

        ids_v = ids_vmem[:, :]
        in_range = (ids_v >= offset) & (ids_v < offset + V)
        masked = jnp.where(in_range, out_ref[:, :], 0.0)
        out_ref[:, :] = masked
        send_ref[:, :] = masked.astype(jnp.bfloat16)

        pl.semaphore_wait(barrier, 1)

        rdma = pltpu.make_async_remote_copy(
            src_ref=send_ref,
            dst_ref=recv_ref,
            send_sem=send_sem,
            recv_sem=recv_sem,
            device_id=partner,
            device_id_type=pl.DeviceIdType.MESH,
        )
        rdma.start()
        rdma.wait()

        out_ref[:, :] = out_ref[:, :] + recv_ref[:, :].astype(jnp.float32)

    ids2 = ids.reshape(T, 1)
    return pl.pallas_call(
        body,
        out_shape=jax.ShapeDtypeStruct((T, D), jnp.float32),
        in_specs=[
            pl.BlockSpec(memory_space=pltpu.SMEM),
            pl.BlockSpec(memory_space=pltpu.VMEM),
            pl.BlockSpec(memory_space=pltpu.ANY),
        ],
        out_specs=pl.BlockSpec(memory_space=pltpu.VMEM),
        scratch_shapes=[
            pltpu.VMEM((T, D), jnp.bfloat16),
            pltpu.VMEM((T, D), jnp.bfloat16),
            pltpu.SemaphoreType.DMA,
            pltpu.SemaphoreType.DMA,
            pltpu.SemaphoreType.DMA,
        ],
        compiler_params=pltpu.CompilerParams(collective_id=0),
    )(ids2, ids2, E)


# baseline (device time: 131143 ns/iter reference)
import jax
import jax.numpy as jnp
from jax import lax
from jax.experimental import pallas as pl
from jax.experimental.pallas import tpu as pltpu

T = 2048
D = 1024


def kernel(ids, E):
    V = E.shape[0]

    def body(ids_smem, ids_vmem, e_hbm, out_ref, send_ref, recv_ref,
             gather_sem, send_sem, recv_sem):
        my_x = lax.axis_index("x")
        my_y = lax.axis_index("y")
        my_z = lax.axis_index("z")
        partner = (1 - my_x, my_y, my_z)

        barrier = pltpu.get_barrier_semaphore()
        pl.semaphore_signal(barrier, inc=1, device_id=partner,
                            device_id_type=pl.DeviceIdType.MESH)

        offset = my_x * V

        def issue(t, carry):
            local = ids_smem[t] - offset
            c = jnp.clip(local, 0, V - 1)
            pltpu.make_async_copy(
                e_hbm.at[pl.ds(c, 1), :],
                out_ref.at[pl.ds(t, 1), :],
                gather_sem,
            ).start()
            return carry

        lax.fori_loop(0, T, issue, 0)

        def drain(t, carry):
            pltpu.make_async_copy(
                e_hbm.at[pl.ds(0, 1), :],
                out_ref.at[pl.ds(t, 1), :],
                gather_sem,
            ).wait()
            return carry

        lax.fori_loop(0, T, drain, 0)

        ids_v = ids_vmem[:, :]
        in_range = (ids_v >= offset) & (ids_v < offset + V)
        masked = jnp.where(in_range, out_ref[:, :], 0.0)
        out_ref[:, :] = masked
        send_ref[:, :] = masked.astype(jnp.bfloat16)

        pl.semaphore_wait(barrier, 1)

        rdma = pltpu.make_async_remote_copy(
            src_ref=send_ref,
            dst_ref=recv_ref,
            send_sem=send_sem,
            recv_sem=recv_sem,
            device_id=partner,
            device_id_type=pl.DeviceIdType.MESH,
        )
        rdma.start()
        rdma.wait()

        out_ref[:, :] = out_ref[:, :] + recv_ref[:, :].astype(jnp.float32)

    ids2 = ids.reshape(T, 1)
    return pl.pallas_call(
        body,
        out_shape=jax.ShapeDtypeStruct((T, D), jnp.float32),
        in_specs=[
            pl.BlockSpec(memory_space=pltpu.SMEM),
            pl.BlockSpec(memory_space=pltpu.VMEM),
            pl.BlockSpec(memory_space=pl.ANY),
        ],
        out_specs=pl.BlockSpec(memory_space=pltpu.VMEM),
        scratch_shapes=[
            pltpu.VMEM((T, D), jnp.bfloat16),
            pltpu.VMEM((T, D), jnp.bfloat16),
            pltpu.SemaphoreType.DMA,
            pltpu.SemaphoreType.DMA,
            pltpu.SemaphoreType.DMA,
        ],
        compiler_params=pltpu.CompilerParams(collective_id=0),
    )(ids, ids2, E)
